# Initial kernel scaffold; baseline (speedup 1.0000x reference)
import jax
import jax.numpy as jnp
from jax.experimental import pallas as pl


def kernel(xyz, new_xyz, features, query_features, W_mlp, b_mlp, Wq, bq, Wax1, bax1, Wax2, bax2, Waf, baf, Wx, bx):
    raise NotImplementedError("write your pallas kernel here")


# speedup vs baseline: 7.8763x; 7.8763x over previous
import functools
import jax
import jax.numpy as jnp
from jax import lax
from jax.experimental import pallas as pl
from jax.experimental.pallas import tpu as pltpu
from jax.experimental.pallas import tpu_sc as plsc

RADIUS2 = 0.2 * 0.2
NS = 32
MBLK = 128
D = 128


def _sel_body(d2_ref, idx_ref, cnt_ref):
    d2 = d2_ref[0]
    N = d2.shape[1]
    within = d2 < RADIUS2
    rank = within.astype(jnp.float32)
    s = 1
    while s < N:
        rank = rank + jnp.concatenate(
            [jnp.zeros((MBLK, s), jnp.float32), rank[:, :N - s]], axis=1)
        s *= 2
    cnt = rank[:, N - 1:N]
    sel = within & (rank <= float(NS))
    iota_n = lax.broadcasted_iota(jnp.int32, (MBLK, N), 1).astype(jnp.float32)
    li32 = lax.broadcasted_iota(jnp.int32, (1, NS), 1)
    idx0 = jnp.sum(jnp.where(sel & (rank == 1.0), iota_n, 0.0),
                   axis=1, keepdims=True)

    def body(k, acc):
        kf = k.astype(jnp.float32)
        col = jnp.sum(jnp.where(sel & (rank == kf + 1.0), iota_n, 0.0),
                      axis=1, keepdims=True)
        col = jnp.where(cnt >= kf + 1.0, col, idx0)
        return jnp.where(li32 == k, col, acc)

    acc0 = jnp.zeros((MBLK, NS), jnp.float32)
    idx = lax.fori_loop(0, NS, body, acc0)
    b = pl.program_id(0)
    gid = idx + jnp.float32(N) * b.astype(jnp.float32)
    idx_ref[0] = gid.astype(jnp.int32)
    cnt_ref[0] = cnt


def _make_sc_gather(rows, d):
    info = plsc.get_sparse_core_info()
    NC, NSUB = info.num_cores, info.num_subcores
    NW = NC * NSUB
    b_per_w = rows // NW
    CH = 128
    mesh = plsc.VectorSubcoreMesh(core_axis_name="c", subcore_axis_name="s")

    @functools.partial(
        pl.kernel, mesh=mesh,
        out_type=jax.ShapeDtypeStruct((rows, d), jnp.float32),
        scratch_types=[
            pltpu.VMEM((CH,), jnp.int32),
            pltpu.VMEM((CH, d), jnp.float32),
            pltpu.SemaphoreType.DMA,
        ],
    )
    def gather_k(table_hbm, idx_hbm, out_hbm, idx_v, rows_v, sem):
        wid = lax.axis_index("s") * NC + lax.axis_index("c")
        base = wid * b_per_w
        for ci in range(b_per_w // CH):
            off = base + ci * CH
            pltpu.sync_copy(idx_hbm.at[pl.ds(off, CH)], idx_v)
            pltpu.async_copy(table_hbm.at[idx_v], rows_v, sem).wait()
            pltpu.sync_copy(rows_v, out_hbm.at[pl.ds(off, CH)])

    return gather_k


def _agg_body(g_ref, qf_ref, nq_ref, cnt_ref,
              WmlpT_ref, bmlp_ref, WqT_ref, bq_ref,
              Wax1T_ref, bax1_ref, wax2T_ref, bax2_ref,
              WafT_ref, baf_ref, WxT_ref, bx_ref,
              out_ref):
    HI = jax.lax.Precision.HIGHEST
    g = g_ref[...]
    S = g.shape[0]
    f = g[:, :64]
    gxyz = g[:, 64:67]
    qf = qf_ref[...]
    nq = nq_ref[...]
    cnt = cnt_ref[...]

    q = jax.nn.relu(jnp.dot(qf, WqT_ref[...], precision=HI) + bq_ref[...])

    def expand(x):
        return jnp.broadcast_to(x[:, None, :], (MBLK, NS, x.shape[1])
                                ).reshape(S, x.shape[1])

    qexp = expand(q)
    nqexp = expand(nq)
    gxd = gxyz - nqexp

    fd = jax.nn.relu(jnp.dot(qexp - f, WafT_ref[...], precision=HI)
                     + baf_ref[...])

    def dot3(lhs, rhs):
        return (lhs[:, 0:1] * rhs[0:1, :] + lhs[:, 1:2] * rhs[1:2, :]
                + lhs[:, 2:3] * rhs[2:3, :])

    h = jax.nn.relu(dot3(gxd, Wax1T_ref[...]) + bax1_ref[...])
    sd = jax.nn.relu(jnp.dot(h, wax2T_ref[...], precision=HI)
                     + bax2_ref[...])
    dd = jnp.sum(gxd * gxd, axis=1, keepdims=True)
    recip = 1.0 / (jnp.sqrt(dd) + 1e-8)
    P = jax.nn.relu(jnp.dot(f, WmlpT_ref[...], precision=HI) + bmlp_ref[...])

    alpha = recip * fd * sd
    num = jnp.sum((alpha * P).reshape(MBLK, NS, 128), axis=1)
    den = jnp.sum(recip.reshape(MBLK, NS, 1), axis=1)
    out_feat = jnp.where(cnt > 0.0, num / den, 0.0)

    X = jax.nn.relu(dot3(gxd, WxT_ref[...]) + bx_ref[...])
    umax = jnp.max(X.reshape(MBLK, NS, 32), axis=1)
    out_ref[...] = jnp.concatenate([umax, out_feat], axis=1)


@jax.jit
def kernel(xyz, new_xyz, features, query_features,
           W_mlp, b_mlp, Wq, bq, Wax1, bax1, Wax2, bax2, Waf, baf, Wx, bx):
    B, N, _ = xyz.shape
    M = new_xyz.shape[1]

    dist2 = (jnp.sum(new_xyz ** 2, axis=-1)[:, :, None]
             + jnp.sum(xyz ** 2, axis=-1)[:, None, :]
             - 2.0 * jnp.einsum('bmd,bnd->bmn', new_xyz, xyz))

    idx, cnt = pl.pallas_call(
        _sel_body,
        grid=(B, M // MBLK),
        in_specs=[pl.BlockSpec((1, MBLK, N), lambda b, mb: (b, mb, 0))],
        out_specs=[
            pl.BlockSpec((1, MBLK, NS), lambda b, mb: (b, mb, 0)),
            pl.BlockSpec((1, MBLK, 1), lambda b, mb: (b, mb, 0)),
        ],
        out_shape=[
            jax.ShapeDtypeStruct((B, M, NS), jnp.int32),
            jax.ShapeDtypeStruct((B, M, 1), jnp.float32),
        ],
    )(dist2)

    featT = jnp.transpose(features, (0, 2, 1))
    table = jnp.concatenate(
        [featT, xyz, jnp.zeros((B, N, D - 67), jnp.float32)], axis=-1
    ).reshape(B * N, D)
    rows = B * M * NS
    g = _make_sc_gather(rows, D)(table, idx.reshape(rows))

    qfT = jnp.transpose(query_features, (0, 2, 1)).reshape(B * M, 128)
    nqf = new_xyz.reshape(B * M, 3)
    cntf = cnt.reshape(B * M, 1)
    nblk = B * M // MBLK
    wspec = lambda shape: pl.BlockSpec(shape, lambda i: (0, 0))
    out = pl.pallas_call(
        _agg_body,
        grid=(nblk,),
        in_specs=[
            pl.BlockSpec((MBLK * NS, D), lambda i: (i, 0)),
            pl.BlockSpec((MBLK, 128), lambda i: (i, 0)),
            pl.BlockSpec((MBLK, 3), lambda i: (i, 0)),
            pl.BlockSpec((MBLK, 1), lambda i: (i, 0)),
            wspec((64, 128)), wspec((1, 128)),
            wspec((128, 64)), wspec((1, 64)),
            wspec((3, 32)), wspec((1, 32)),
            wspec((32, 1)), wspec((1, 1)),
            wspec((64, 1)), wspec((1, 1)),
            wspec((3, 32)), wspec((1, 32)),
        ],
        out_specs=pl.BlockSpec((MBLK, 160), lambda i: (i, 0)),
        out_shape=jax.ShapeDtypeStruct((B * M, 160), jnp.float32),
    )(
        g, qfT, nqf, cntf,
        W_mlp.T, b_mlp[None, :], Wq.T, bq[None, :],
        Wax1.T, bax1[None, :], Wax2.T, bax2[None, :],
        Waf.T, baf[None, :], Wx.T, bx[None, :],
    )
    out3 = jnp.transpose(out.reshape(B, M, 160), (0, 2, 1))
    return (new_xyz, out3)


# speedup vs baseline: 8.1352x; 1.0329x over previous
import functools
import jax
import jax.numpy as jnp
from jax import lax
from jax.experimental import pallas as pl
from jax.experimental.pallas import tpu as pltpu
from jax.experimental.pallas import tpu_sc as plsc

RADIUS2 = 0.2 * 0.2
NS = 32
MBLK = 128
D = 128


def _sel_body(d2_ref, idx_ref, cnt_ref):
    d2 = d2_ref[0]
    N = d2.shape[1]
    within = d2 < RADIUS2
    rank = within.astype(jnp.float32)
    s = 1
    while s < N:
        rank = rank + jnp.concatenate(
            [jnp.zeros((MBLK, s), jnp.float32), rank[:, :N - s]], axis=1)
        s *= 2
    cnt = rank[:, N - 1:N]
    sel = within & (rank <= float(NS))
    iota_n = lax.broadcasted_iota(jnp.int32, (MBLK, N), 1).astype(jnp.float32)
    li32 = lax.broadcasted_iota(jnp.int32, (1, NS), 1)
    idx0 = jnp.sum(jnp.where(sel & (rank == 1.0), iota_n, 0.0),
                   axis=1, keepdims=True)

    def body(k, acc):
        kf = k.astype(jnp.float32)
        col = jnp.sum(jnp.where(sel & (rank == kf + 1.0), iota_n, 0.0),
                      axis=1, keepdims=True)
        col = jnp.where(cnt >= kf + 1.0, col, idx0)
        return jnp.where(li32 == k, col, acc)

    acc0 = jnp.zeros((MBLK, NS), jnp.float32)
    idx = lax.fori_loop(0, NS, body, acc0)
    b = pl.program_id(0)
    gid = idx + jnp.float32(N) * b.astype(jnp.float32)
    idx_ref[0] = gid.astype(jnp.int32)
    cnt_ref[0] = cnt


def _make_sc_gather(rows, d):
    info = plsc.get_sparse_core_info()
    NC, NSUB = info.num_cores, info.num_subcores
    NW = NC * NSUB
    b_per_w = rows // NW
    CH = 512
    mesh = plsc.VectorSubcoreMesh(core_axis_name="c", subcore_axis_name="s")

    @functools.partial(
        pl.kernel, mesh=mesh,
        out_type=jax.ShapeDtypeStruct((rows, d), jnp.float32),
        scratch_types=[
            pltpu.VMEM((CH,), jnp.int32),
            pltpu.VMEM((CH, d), jnp.float32),
            pltpu.SemaphoreType.DMA,
        ],
    )
    def gather_k(table_hbm, idx_hbm, out_hbm, idx_v, rows_v, sem):
        wid = lax.axis_index("s") * NC + lax.axis_index("c")
        base = wid * b_per_w
        for ci in range(b_per_w // CH):
            off = base + ci * CH
            pltpu.sync_copy(idx_hbm.at[pl.ds(off, CH)], idx_v)
            pltpu.async_copy(table_hbm.at[idx_v], rows_v, sem).wait()
            pltpu.sync_copy(rows_v, out_hbm.at[pl.ds(off, CH)])

    return gather_k


def _agg_body(g_ref, qf_ref, nq_ref, cnt_ref,
              WmlpT_ref, bmlp_ref, WqT_ref, bq_ref,
              Wax1T_ref, bax1_ref, wax2T_ref, bax2_ref,
              WafT_ref, baf_ref, WxT_ref, bx_ref,
              out_ref):
    HI = jax.lax.Precision.HIGHEST
    g = g_ref[...]
    S = g.shape[0]
    f = g[:, :64]
    gxyz = g[:, 64:67]
    qf = qf_ref[...]
    nq = nq_ref[...]
    cnt = cnt_ref[...]

    q = jax.nn.relu(jnp.dot(qf, WqT_ref[...], precision=HI) + bq_ref[...])

    def expand(x):
        return jnp.broadcast_to(x[:, None, :], (MBLK, NS, x.shape[1])
                                ).reshape(S, x.shape[1])

    qexp = expand(q)
    nqexp = expand(nq)
    gxd = gxyz - nqexp

    fd = jax.nn.relu(jnp.dot(qexp - f, WafT_ref[...], precision=HI)
                     + baf_ref[...])

    def dot3(lhs, rhs):
        return (lhs[:, 0:1] * rhs[0:1, :] + lhs[:, 1:2] * rhs[1:2, :]
                + lhs[:, 2:3] * rhs[2:3, :])

    h = jax.nn.relu(dot3(gxd, Wax1T_ref[...]) + bax1_ref[...])
    sd = jax.nn.relu(jnp.dot(h, wax2T_ref[...], precision=HI)
                     + bax2_ref[...])
    dd = jnp.sum(gxd * gxd, axis=1, keepdims=True)
    recip = 1.0 / (jnp.sqrt(dd) + 1e-8)
    P = jax.nn.relu(jnp.dot(f, WmlpT_ref[...], precision=HI) + bmlp_ref[...])

    alpha = recip * fd * sd
    num = jnp.sum((alpha * P).reshape(MBLK, NS, 128), axis=1)
    den = jnp.sum(recip.reshape(MBLK, NS, 1), axis=1)
    out_feat = jnp.where(cnt > 0.0, num / den, 0.0)

    X = jax.nn.relu(dot3(gxd, WxT_ref[...]) + bx_ref[...])
    umax = jnp.max(X.reshape(MBLK, NS, 32), axis=1)
    out_ref[...] = jnp.concatenate([umax, out_feat], axis=1)


@jax.jit
def kernel(xyz, new_xyz, features, query_features,
           W_mlp, b_mlp, Wq, bq, Wax1, bax1, Wax2, bax2, Waf, baf, Wx, bx):
    B, N, _ = xyz.shape
    M = new_xyz.shape[1]

    dist2 = (jnp.sum(new_xyz ** 2, axis=-1)[:, :, None]
             + jnp.sum(xyz ** 2, axis=-1)[:, None, :]
             - 2.0 * jnp.einsum('bmd,bnd->bmn', new_xyz, xyz))

    idx, cnt = pl.pallas_call(
        _sel_body,
        grid=(B, M // MBLK),
        in_specs=[pl.BlockSpec((1, MBLK, N), lambda b, mb: (b, mb, 0))],
        out_specs=[
            pl.BlockSpec((1, MBLK, NS), lambda b, mb: (b, mb, 0)),
            pl.BlockSpec((1, MBLK, 1), lambda b, mb: (b, mb, 0)),
        ],
        out_shape=[
            jax.ShapeDtypeStruct((B, M, NS), jnp.int32),
            jax.ShapeDtypeStruct((B, M, 1), jnp.float32),
        ],
    )(dist2)

    featT = jnp.transpose(features, (0, 2, 1))
    table = jnp.concatenate(
        [featT, xyz, jnp.zeros((B, N, D - 67), jnp.float32)], axis=-1
    ).reshape(B * N, D)
    rows = B * M * NS
    g = _make_sc_gather(rows, D)(table, idx.reshape(rows))

    qfT = jnp.transpose(query_features, (0, 2, 1)).reshape(B * M, 128)
    nqf = new_xyz.reshape(B * M, 3)
    cntf = cnt.reshape(B * M, 1)
    nblk = B * M // MBLK
    wspec = lambda shape: pl.BlockSpec(shape, lambda i: (0, 0))
    out = pl.pallas_call(
        _agg_body,
        grid=(nblk,),
        in_specs=[
            pl.BlockSpec((MBLK * NS, D), lambda i: (i, 0)),
            pl.BlockSpec((MBLK, 128), lambda i: (i, 0)),
            pl.BlockSpec((MBLK, 3), lambda i: (i, 0)),
            pl.BlockSpec((MBLK, 1), lambda i: (i, 0)),
            wspec((64, 128)), wspec((1, 128)),
            wspec((128, 64)), wspec((1, 64)),
            wspec((3, 32)), wspec((1, 32)),
            wspec((32, 1)), wspec((1, 1)),
            wspec((64, 1)), wspec((1, 1)),
            wspec((3, 32)), wspec((1, 32)),
        ],
        out_specs=pl.BlockSpec((MBLK, 160), lambda i: (i, 0)),
        out_shape=jax.ShapeDtypeStruct((B * M, 160), jnp.float32),
    )(
        g, qfT, nqf, cntf,
        W_mlp.T, b_mlp[None, :], Wq.T, bq[None, :],
        Wax1.T, bax1[None, :], Wax2.T, bax2[None, :],
        Waf.T, baf[None, :], Wx.T, bx[None, :],
    )
    out3 = jnp.transpose(out.reshape(B, M, 160), (0, 2, 1))
    return (new_xyz, out3)
